# baseline (device time: 1192480 ns/iter reference)
import jax
import jax.numpy as jnp
from jax import lax
from jax.experimental import pallas as pl
from jax.experimental.pallas import tpu as pltpu

N_DEV = 4
ROWS = 4096
COLS = 1024
MAXC = 1152


def _body(send_ref, cnt_ref, data_out, cnt_out,
          data_send, data_recv, cnt_send, cnt_recv, local_sem):
    me = lax.axis_index("i")

    barrier = pltpu.get_barrier_semaphore()
    for d in range(1, N_DEV):
        pl.semaphore_signal(
            barrier, inc=1,
            device_id=((me + d) % N_DEV,),
            device_id_type=pl.DeviceIdType.MESH,
        )
    pl.semaphore_wait(barrier, N_DEV - 1)

    own = pltpu.make_async_copy(send_ref.at[me], data_out.at[0], local_sem)
    own.start()
    cnt_out[0, :, :] = cnt_ref[:, :]

    rdmas = []
    for d in range(1, N_DEV):
        tgt = (me + d) % N_DEV
        dr = pltpu.make_async_remote_copy(
            src_ref=send_ref.at[tgt],
            dst_ref=data_out.at[d],
            send_sem=data_send.at[d],
            recv_sem=data_recv.at[d],
            device_id=(tgt,),
            device_id_type=pl.DeviceIdType.MESH,
        )
        cr = pltpu.make_async_remote_copy(
            src_ref=cnt_ref,
            dst_ref=cnt_out.at[d],
            send_sem=cnt_send.at[d],
            recv_sem=cnt_recv.at[d],
            device_id=(tgt,),
            device_id_type=pl.DeviceIdType.MESH,
        )
        dr.start()
        cr.start()
        rdmas.append((dr, cr))

    for dr, cr in rdmas:
        dr.wait_send()
        cr.wait_send()
    for dr, cr in rdmas:
        dr.wait_recv()
        cr.wait_recv()
    own.wait()


def kernel(x, dest):
    counts = jnp.sum(
        dest[:, None] == jnp.arange(N_DEV, dtype=dest.dtype)[None, :], axis=0
    ).astype(jnp.int32)
    offsets = jnp.concatenate(
        [jnp.zeros((1,), jnp.int32), jnp.cumsum(counts)[:-1]]
    )
    order = jnp.argsort(dest, stable=True)
    x_sorted = jnp.take(x, order, axis=0)
    idx = offsets[:, None] + jnp.arange(MAXC, dtype=jnp.int32)[None, :]
    send_buf = jnp.take(x_sorted, idx, axis=0, mode="fill", fill_value=0.0)
    cnt_in = jnp.zeros((8, 128), jnp.int32).at[0, :N_DEV].set(counts)

    data_out, cnt_out = pl.pallas_call(
        _body,
        out_shape=[
            jax.ShapeDtypeStruct((N_DEV, MAXC, COLS), jnp.float32),
            jax.ShapeDtypeStruct((N_DEV, 8, 128), jnp.int32),
        ],
        in_specs=[
            pl.BlockSpec(memory_space=pltpu.VMEM),
            pl.BlockSpec(memory_space=pltpu.VMEM),
        ],
        out_specs=[
            pl.BlockSpec(memory_space=pltpu.VMEM),
            pl.BlockSpec(memory_space=pltpu.VMEM),
        ],
        scratch_shapes=[
            pltpu.SemaphoreType.DMA((N_DEV,)),
            pltpu.SemaphoreType.DMA((N_DEV,)),
            pltpu.SemaphoreType.DMA((N_DEV,)),
            pltpu.SemaphoreType.DMA((N_DEV,)),
            pltpu.SemaphoreType.DMA,
        ],
        compiler_params=pltpu.CompilerParams(collective_id=0),
    )(send_buf, cnt_in)

    me = lax.axis_index("i")
    slots = (me - jnp.arange(N_DEV)) % N_DEV
    cnt_all = cnt_out[slots, 0, me]
    ends = jnp.cumsum(cnt_all)
    starts = ends - cnt_all
    k = jnp.arange(ROWS)
    s_k = jnp.searchsorted(ends, k, side="right")
    j_k = k - starts[s_k]
    return data_out[slots[s_k], j_k]


# device time: 161319 ns/iter; 7.3921x vs baseline; 7.3921x over previous
import jax
import jax.numpy as jnp
from jax import lax
from jax.experimental import pallas as pl
from jax.experimental.pallas import tpu as pltpu

N_DEV = 4
ROWS = 4096
COLS = 1024
MAXC = 1152


def _body(x_ref, pos_ref, mycnt_ref, cnt_in_ref,
          out_ref,
          send_buf, recv_buf, cnt_recv, cnt_smem,
          data_send, data_recv, cnt_send, cnt_recv_sems, copy_sem):
    me = lax.axis_index("i")

    barrier = pltpu.get_barrier_semaphore()
    for d in range(1, N_DEV):
        pl.semaphore_signal(
            barrier, inc=1,
            device_id=((me + d) % N_DEV,),
            device_id_type=pl.DeviceIdType.MESH,
        )

    def scatter(j, carry):
        send_buf[pos_ref[j]] = x_ref[j]
        return carry

    lax.fori_loop(0, ROWS, scatter, 0, unroll=8)

    pl.semaphore_wait(barrier, N_DEV - 1)

    rdmas = []
    for d in range(1, N_DEV):
        tgt = (me + d) % N_DEV
        cr = pltpu.make_async_remote_copy(
            src_ref=cnt_in_ref,
            dst_ref=cnt_recv.at[d],
            send_sem=cnt_send.at[d],
            recv_sem=cnt_recv_sems.at[d],
            device_id=(tgt,),
            device_id_type=pl.DeviceIdType.MESH,
        )
        dr = pltpu.make_async_remote_copy(
            src_ref=send_buf.at[pl.ds(tgt * MAXC, MAXC)],
            dst_ref=recv_buf.at[d - 1],
            send_sem=data_send.at[d],
            recv_sem=data_recv.at[d],
            device_id=(tgt,),
            device_id_type=pl.DeviceIdType.MESH,
        )
        cr.start()
        dr.start()
        rdmas.append((dr, cr))

    for dr, cr in rdmas:
        cr.wait_recv()
    cnt_copy = pltpu.make_async_copy(cnt_recv, cnt_smem, copy_sem)
    cnt_copy.start()
    cnt_copy.wait()

    for dr, cr in rdmas:
        dr.wait_recv()

    start = jnp.int32(0)
    for s in range(N_DEV):
        d = (me - s) % N_DEV
        cnt_s = jnp.where(d == 0, mycnt_ref[s], cnt_smem[d, 0, me])

        @pl.when(d == 0)
        def _(start=start):
            cp = pltpu.make_async_copy(
                send_buf.at[pl.ds(s * MAXC, MAXC)],
                out_ref.at[pl.ds(start, MAXC)],
                copy_sem,
            )
            cp.start()
            cp.wait()

        @pl.when(d != 0)
        def _(start=start, d=d):
            cp = pltpu.make_async_copy(
                recv_buf.at[d - 1],
                out_ref.at[pl.ds(start, MAXC)],
                copy_sem,
            )
            cp.start()
            cp.wait()

        start = start + cnt_s

    for dr, cr in rdmas:
        dr.wait_send()
        cr.wait_send()


def kernel(x, dest):
    onehot = (dest[:, None] == jnp.arange(N_DEV, dtype=dest.dtype)).astype(
        jnp.int32
    )
    csum = jnp.cumsum(onehot, axis=0)
    rank = jnp.sum(csum * onehot, axis=1) - 1
    counts = csum[-1]
    pos = dest.astype(jnp.int32) * MAXC + rank
    cnt_in = jnp.zeros((8, 128), jnp.int32).at[0, :N_DEV].set(counts)

    x3 = x.reshape(ROWS, 8, 128)

    out = pl.pallas_call(
        _body,
        out_shape=jax.ShapeDtypeStruct((ROWS + MAXC, 8, 128), jnp.float32),
        in_specs=[
            pl.BlockSpec(memory_space=pltpu.VMEM),
            pl.BlockSpec(memory_space=pltpu.SMEM),
            pl.BlockSpec(memory_space=pltpu.SMEM),
            pl.BlockSpec(memory_space=pltpu.VMEM),
        ],
        out_specs=pl.BlockSpec(memory_space=pl.ANY),
        scratch_shapes=[
            pltpu.VMEM((N_DEV * MAXC, 8, 128), jnp.float32),
            pltpu.VMEM((N_DEV - 1, MAXC, 8, 128), jnp.float32),
            pltpu.VMEM((N_DEV, 8, 128), jnp.int32),
            pltpu.SMEM((N_DEV, 8, 128), jnp.int32),
            pltpu.SemaphoreType.DMA((N_DEV,)),
            pltpu.SemaphoreType.DMA((N_DEV,)),
            pltpu.SemaphoreType.DMA((N_DEV,)),
            pltpu.SemaphoreType.DMA((N_DEV,)),
            pltpu.SemaphoreType.DMA,
        ],
        compiler_params=pltpu.CompilerParams(
            collective_id=0,
            vmem_limit_bytes=100 * 1024 * 1024,
        ),
    )(x3, pos, counts, cnt_in)

    return out[:ROWS].reshape(ROWS, COLS)
